# baseline (speedup 1.0000x reference)
import functools

import jax
import jax.numpy as jnp
from jax.experimental import pallas as pl
from jax.experimental.pallas import tpu as pltpu

LANES = 128
TARGET_BLOCK_BYTES = 4 * 1024 * 1024
VMEM_LIMIT_BYTES = 48 * 1024 * 1024


def _round_up(x: int, m: int) -> int:
    return (x + m - 1) // m * m


def _fold_rows(x):
    r, w = x.shape
    return jnp.sum(x.reshape(r // 8, 8, w), axis=0)


def _scalar_f1(tp, sp, st, beta2, eps):
    epsf = jnp.float32(eps)
    fn = sp - tp
    fp = st - tp
    p = tp / (tp + fp + epsf)
    r = tp / (tp + fn + epsf)
    f1 = (1.0 + beta2) * (p * r) / (beta2 * p + r + epsf)
    f1 = jnp.where(jnp.isnan(f1), jnp.zeros_like(f1), f1)
    return (1.0 - f1).astype(jnp.float32)


MANUAL_TILE_R = 512
MANUAL_DEPTH = 6


def _manual_ring_kernel(*refs, n_steps: int, tile_r: int, has_tail: bool,
                        beta2: float, eps: float):
    if has_tail:
        tail_ref, yp_hbm, yt_hbm, out_ref, ypb, ytb, tp_ref, sp_ref, st_ref, sems = refs
    else:
        yp_hbm, yt_hbm, out_ref, ypb, ytb, tp_ref, sp_ref, st_ref, sems = refs
        tail_ref = None
    K = MANUAL_DEPTH

    def copy(i, slot):
        rows = pl.ds(i * tile_r, tile_r)
        return (pltpu.make_async_copy(yp_hbm.at[rows, :], ypb.at[slot],
                                      sems.at[0, slot]),
                pltpu.make_async_copy(yt_hbm.at[rows, :], ytb.at[slot],
                                      sems.at[1, slot]))

    for i in range(min(K, n_steps)):
        for c in copy(i, i):
            c.start()

    tp_ref[...] = jnp.zeros_like(tp_ref)
    sp_ref[...] = jnp.zeros_like(sp_ref)
    st_ref[...] = jnp.zeros_like(st_ref)

    def body(i, _):
        slot = jax.lax.rem(i, K)
        cp, ct = copy(i, slot)
        cp.wait()
        ct.wait()
        yp = ypb[slot]
        yt = ytb[slot]
        tp_ref[...] += _fold_rows(yt * yp)
        sp_ref[...] += _fold_rows(yp)
        st_ref[...] += _fold_rows(yt)

        nxt = i + K

        @pl.when(nxt < n_steps)
        def _prefetch():
            np_, nt = copy(nxt, slot)
            np_.start()
            nt.start()
        return ()

    jax.lax.fori_loop(0, n_steps, body, (), unroll=False)

    tp = jnp.sum(tp_ref[...])
    sp = jnp.sum(sp_ref[...])
    st = jnp.sum(st_ref[...])
    if has_tail:
        tp = tp + tail_ref[0]
        sp = sp + tail_ref[1]
        st = st + tail_ref[2]
    out_ref[0, 0] = _scalar_f1(tp, sp, st, beta2, eps)


def _soft_f1_manual(yp2: jax.Array, yt2: jax.Array, tails, beta2, eps):
    R, W = yp2.shape
    tile_r = MANUAL_TILE_R
    n_steps = R // tile_r
    has_tail = tails is not None

    kern = functools.partial(
        _manual_ring_kernel, n_steps=n_steps, tile_r=tile_r,
        has_tail=has_tail, beta2=beta2, eps=eps)

    data_specs = [pl.BlockSpec(memory_space=pl.ANY),
                  pl.BlockSpec(memory_space=pl.ANY)]
    if has_tail:
        in_specs = [pl.BlockSpec(memory_space=pltpu.SMEM)] + data_specs
        operands = (tails, yp2, yt2)
    else:
        in_specs = data_specs
        operands = (yp2, yt2)

    in_bytes = yp2.size * yp2.dtype.itemsize + yt2.size * yt2.dtype.itemsize
    out = pl.pallas_call(
        kern,
        out_shape=jax.ShapeDtypeStruct((1, 1), jnp.float32),
        in_specs=in_specs,
        out_specs=pl.BlockSpec(memory_space=pltpu.SMEM),
        scratch_shapes=[
            pltpu.VMEM((MANUAL_DEPTH, tile_r, W), jnp.float32),
            pltpu.VMEM((MANUAL_DEPTH, tile_r, W), jnp.float32),
            pltpu.VMEM((8, W), jnp.float32),
            pltpu.VMEM((8, W), jnp.float32),
            pltpu.VMEM((8, W), jnp.float32),
            pltpu.SemaphoreType.DMA((2, MANUAL_DEPTH)),
        ],
        compiler_params=pltpu.CompilerParams(
            vmem_limit_bytes=VMEM_LIMIT_BYTES),
        cost_estimate=pl.CostEstimate(
            flops=4 * yp2.size, transcendentals=0,
            bytes_accessed=in_bytes + 4),
    )(*operands)
    return out[0, 0]


def _soft_f1_kernel(*refs,
                    tile_r: int, n_steps: int, rows_total: int,
                    full_blocks: int, any_masked: bool, has_tail: bool,
                    beta2: float, eps: float):
    if has_tail:
        tail_ref, yp_ref, yt_ref, out_ref, tp_ref, sp_ref, st_ref = refs
    else:
        yp_ref, yt_ref, out_ref, tp_ref, sp_ref, st_ref = refs
        tail_ref = None
    s = pl.program_id(0)

    @pl.when(s == 0)
    def _init():
        tp_ref[...] = jnp.zeros_like(tp_ref)
        sp_ref[...] = jnp.zeros_like(sp_ref)
        st_ref[...] = jnp.zeros_like(st_ref)

    yp = yp_ref[...].astype(jnp.float32)
    yt = yt_ref[...].astype(jnp.float32)

    def accumulate(ypv, ytv):
        tp_ref[...] += _fold_rows(ytv * ypv)
        sp_ref[...] += _fold_rows(ypv)
        st_ref[...] += _fold_rows(ytv)

    if any_masked:
        @pl.when(s >= full_blocks)
        def _edge():
            row = s * tile_r + jax.lax.broadcasted_iota(
                jnp.int32, yp.shape, 0)
            valid = row < rows_total
            accumulate(jnp.where(valid, yp, 0.0), jnp.where(valid, yt, 0.0))

        @pl.when(s < full_blocks)
        def _full():
            accumulate(yp, yt)
    else:
        accumulate(yp, yt)

    @pl.when(s == n_steps - 1)
    def _finish():
        tp = jnp.sum(tp_ref[...])
        sp = jnp.sum(sp_ref[...])
        st = jnp.sum(st_ref[...])
        if has_tail:
            tp = tp + tail_ref[0]
            sp = sp + tail_ref[1]
            st = st + tail_ref[2]
        epsf = jnp.float32(eps)
        fn = sp - tp
        fp = st - tp
        p = tp / (tp + fp + epsf)
        r = tp / (tp + fn + epsf)
        f1 = (1.0 + beta2) * (p * r) / (beta2 * p + r + epsf)
        f1 = jnp.where(jnp.isnan(f1), jnp.zeros_like(f1), f1)
        out_ref[0, 0] = (1.0 - f1).astype(jnp.float32)


def _soft_f1_pallas(yp2: jax.Array, yt2: jax.Array, tails, beta2, eps):
    R, W = yp2.shape
    tile_r = max(8, min(_round_up(R, 8), TARGET_BLOCK_BYTES // (4 * W)))
    tile_r = _round_up(tile_r, 8)

    n_blocks = pl.cdiv(R, tile_r)
    full_blocks = R // tile_r
    any_masked = n_blocks > full_blocks

    has_tail = tails is not None
    kern = functools.partial(
        _soft_f1_kernel, tile_r=tile_r, n_steps=n_blocks,
        rows_total=R, full_blocks=full_blocks, any_masked=any_masked,
        has_tail=has_tail, beta2=beta2, eps=eps)

    in_map = lambda s: (s, 0)
    in_bytes = yp2.size * yp2.dtype.itemsize + yt2.size * yt2.dtype.itemsize

    data_specs = [pl.BlockSpec((tile_r, W), in_map),
                  pl.BlockSpec((tile_r, W), in_map)]
    if has_tail:
        in_specs = [pl.BlockSpec(memory_space=pltpu.SMEM)] + data_specs
        operands = (tails, yp2, yt2)
    else:
        in_specs = data_specs
        operands = (yp2, yt2)

    out = pl.pallas_call(
        kern,
        out_shape=jax.ShapeDtypeStruct((1, 1), jnp.float32),
        grid=(n_blocks,),
        in_specs=in_specs,
        out_specs=pl.BlockSpec(memory_space=pltpu.SMEM),
        scratch_shapes=[pltpu.VMEM((8, W), jnp.float32)] * 3,
        compiler_params=pltpu.CompilerParams(
            dimension_semantics=("arbitrary",),
            vmem_limit_bytes=VMEM_LIMIT_BYTES),
        cost_estimate=pl.CostEstimate(
            flops=4 * yp2.size, transcendentals=0,
            bytes_accessed=in_bytes + 4),
    )(*operands)

    return out[0, 0]


def kernel(y_pred: jax.Array, y_true: jax.Array) -> jax.Array:
    beta2 = 1.0
    eps = 1e-6

    n = y_pred.size
    yp_flat = y_pred.reshape(-1)
    yt_flat = y_true.reshape(-1)

    n_main = (n // LANES) * LANES

    if n_main < n:
        ypt = yp_flat[n_main:].astype(jnp.float32)
        ytt = yt_flat[n_main:].astype(jnp.float32)
        tails = jnp.stack(
            [jnp.sum(ytt * ypt), jnp.sum(ypt), jnp.sum(ytt)])
    else:
        tails = None

    if n_main == 0:
        tp, sp, st = tails[0], tails[1], tails[2]
        epsf = jnp.float32(eps)
        fn = sp - tp
        fp = st - tp
        p = tp / (tp + fp + epsf)
        r = tp / (tp + fn + epsf)
        f1 = (1.0 + beta2) * (p * r) / (beta2 * p + r + epsf)
        f1 = jnp.where(jnp.isnan(f1), jnp.zeros_like(f1), f1)
        return (1.0 - f1).astype(jnp.float32)

    yp_main = yp_flat if n_main == n else yp_flat[:n_main]
    yt_main = yt_flat if n_main == n else yt_flat[:n_main]
    W = LANES
    for w in (1024, 512, 256):
        if n_main % w == 0:
            W = w
            break
    R = n_main // W
    yp2 = yp_main.reshape(R, W).astype(jnp.float32)
    yt2 = yt_main.reshape(R, W).astype(jnp.float32)
    if R % MANUAL_TILE_R == 0 and R // MANUAL_TILE_R >= MANUAL_DEPTH:
        return _soft_f1_manual(yp2, yt2, tails, beta2, eps)
    return _soft_f1_pallas(yp2, yt2, tails, beta2, eps)


# speedup vs baseline: 1.0071x; 1.0071x over previous
import functools

import jax
import jax.numpy as jnp
from jax.experimental import pallas as pl
from jax.experimental.pallas import tpu as pltpu

LANES = 128
TARGET_BLOCK_BYTES = 4 * 1024 * 1024
VMEM_LIMIT_BYTES = 48 * 1024 * 1024


def _round_up(x: int, m: int) -> int:
    return (x + m - 1) // m * m


def _fold_rows(x):
    r, w = x.shape
    return jnp.sum(x.reshape(r // 8, 8, w), axis=0)


def _scalar_f1(tp, sp, st, beta2, eps):
    epsf = jnp.float32(eps)
    fn = sp - tp
    fp = st - tp
    p = tp / (tp + fp + epsf)
    r = tp / (tp + fn + epsf)
    f1 = (1.0 + beta2) * (p * r) / (beta2 * p + r + epsf)
    f1 = jnp.where(jnp.isnan(f1), jnp.zeros_like(f1), f1)
    return (1.0 - f1).astype(jnp.float32)


MANUAL_TILE_R = 1024
MANUAL_DEPTH = 4


def _manual_ring_kernel(*refs, n_steps: int, tile_r: int, has_tail: bool,
                        beta2: float, eps: float):
    if has_tail:
        tail_ref, yp_hbm, yt_hbm, out_ref, ypb, ytb, tp_ref, sp_ref, st_ref, sems = refs
    else:
        yp_hbm, yt_hbm, out_ref, ypb, ytb, tp_ref, sp_ref, st_ref, sems = refs
        tail_ref = None
    K = MANUAL_DEPTH

    def copy(i, slot):
        rows = pl.ds(i * tile_r, tile_r)
        return (pltpu.make_async_copy(yp_hbm.at[rows, :], ypb.at[slot],
                                      sems.at[0, slot]),
                pltpu.make_async_copy(yt_hbm.at[rows, :], ytb.at[slot],
                                      sems.at[1, slot]))

    for i in range(min(K, n_steps)):
        for c in copy(i, i):
            c.start()

    tp_ref[...] = jnp.zeros_like(tp_ref)
    sp_ref[...] = jnp.zeros_like(sp_ref)
    st_ref[...] = jnp.zeros_like(st_ref)

    def body(i, _):
        slot = jax.lax.rem(i, K)
        cp, ct = copy(i, slot)
        cp.wait()
        ct.wait()
        yp = ypb[slot]
        yt = ytb[slot]
        tp_ref[...] += _fold_rows(yt * yp)
        sp_ref[...] += _fold_rows(yp)
        st_ref[...] += _fold_rows(yt)

        nxt = i + K

        @pl.when(nxt < n_steps)
        def _prefetch():
            np_, nt = copy(nxt, slot)
            np_.start()
            nt.start()
        return ()

    jax.lax.fori_loop(0, n_steps, body, (), unroll=False)

    tp = jnp.sum(tp_ref[...])
    sp = jnp.sum(sp_ref[...])
    st = jnp.sum(st_ref[...])
    if has_tail:
        tp = tp + tail_ref[0]
        sp = sp + tail_ref[1]
        st = st + tail_ref[2]
    out_ref[0, 0] = _scalar_f1(tp, sp, st, beta2, eps)


def _soft_f1_manual(yp2: jax.Array, yt2: jax.Array, tails, beta2, eps):
    R, W = yp2.shape
    tile_r = MANUAL_TILE_R
    n_steps = R // tile_r
    has_tail = tails is not None

    kern = functools.partial(
        _manual_ring_kernel, n_steps=n_steps, tile_r=tile_r,
        has_tail=has_tail, beta2=beta2, eps=eps)

    data_specs = [pl.BlockSpec(memory_space=pl.ANY),
                  pl.BlockSpec(memory_space=pl.ANY)]
    if has_tail:
        in_specs = [pl.BlockSpec(memory_space=pltpu.SMEM)] + data_specs
        operands = (tails, yp2, yt2)
    else:
        in_specs = data_specs
        operands = (yp2, yt2)

    in_bytes = yp2.size * yp2.dtype.itemsize + yt2.size * yt2.dtype.itemsize
    out = pl.pallas_call(
        kern,
        out_shape=jax.ShapeDtypeStruct((1, 1), jnp.float32),
        in_specs=in_specs,
        out_specs=pl.BlockSpec(memory_space=pltpu.SMEM),
        scratch_shapes=[
            pltpu.VMEM((MANUAL_DEPTH, tile_r, W), jnp.float32),
            pltpu.VMEM((MANUAL_DEPTH, tile_r, W), jnp.float32),
            pltpu.VMEM((8, W), jnp.float32),
            pltpu.VMEM((8, W), jnp.float32),
            pltpu.VMEM((8, W), jnp.float32),
            pltpu.SemaphoreType.DMA((2, MANUAL_DEPTH)),
        ],
        compiler_params=pltpu.CompilerParams(
            vmem_limit_bytes=VMEM_LIMIT_BYTES),
        cost_estimate=pl.CostEstimate(
            flops=4 * yp2.size, transcendentals=0,
            bytes_accessed=in_bytes + 4),
    )(*operands)
    return out[0, 0]


def _soft_f1_kernel(*refs,
                    tile_r: int, n_steps: int, rows_total: int,
                    full_blocks: int, any_masked: bool, has_tail: bool,
                    beta2: float, eps: float):
    if has_tail:
        tail_ref, yp_ref, yt_ref, out_ref, tp_ref, sp_ref, st_ref = refs
    else:
        yp_ref, yt_ref, out_ref, tp_ref, sp_ref, st_ref = refs
        tail_ref = None
    s = pl.program_id(0)

    @pl.when(s == 0)
    def _init():
        tp_ref[...] = jnp.zeros_like(tp_ref)
        sp_ref[...] = jnp.zeros_like(sp_ref)
        st_ref[...] = jnp.zeros_like(st_ref)

    yp = yp_ref[...].astype(jnp.float32)
    yt = yt_ref[...].astype(jnp.float32)

    def accumulate(ypv, ytv):
        tp_ref[...] += _fold_rows(ytv * ypv)
        sp_ref[...] += _fold_rows(ypv)
        st_ref[...] += _fold_rows(ytv)

    if any_masked:
        @pl.when(s >= full_blocks)
        def _edge():
            row = s * tile_r + jax.lax.broadcasted_iota(
                jnp.int32, yp.shape, 0)
            valid = row < rows_total
            accumulate(jnp.where(valid, yp, 0.0), jnp.where(valid, yt, 0.0))

        @pl.when(s < full_blocks)
        def _full():
            accumulate(yp, yt)
    else:
        accumulate(yp, yt)

    @pl.when(s == n_steps - 1)
    def _finish():
        tp = jnp.sum(tp_ref[...])
        sp = jnp.sum(sp_ref[...])
        st = jnp.sum(st_ref[...])
        if has_tail:
            tp = tp + tail_ref[0]
            sp = sp + tail_ref[1]
            st = st + tail_ref[2]
        epsf = jnp.float32(eps)
        fn = sp - tp
        fp = st - tp
        p = tp / (tp + fp + epsf)
        r = tp / (tp + fn + epsf)
        f1 = (1.0 + beta2) * (p * r) / (beta2 * p + r + epsf)
        f1 = jnp.where(jnp.isnan(f1), jnp.zeros_like(f1), f1)
        out_ref[0, 0] = (1.0 - f1).astype(jnp.float32)


def _soft_f1_pallas(yp2: jax.Array, yt2: jax.Array, tails, beta2, eps):
    R, W = yp2.shape
    tile_r = max(8, min(_round_up(R, 8), TARGET_BLOCK_BYTES // (4 * W)))
    tile_r = _round_up(tile_r, 8)

    n_blocks = pl.cdiv(R, tile_r)
    full_blocks = R // tile_r
    any_masked = n_blocks > full_blocks

    has_tail = tails is not None
    kern = functools.partial(
        _soft_f1_kernel, tile_r=tile_r, n_steps=n_blocks,
        rows_total=R, full_blocks=full_blocks, any_masked=any_masked,
        has_tail=has_tail, beta2=beta2, eps=eps)

    in_map = lambda s: (s, 0)
    in_bytes = yp2.size * yp2.dtype.itemsize + yt2.size * yt2.dtype.itemsize

    data_specs = [pl.BlockSpec((tile_r, W), in_map),
                  pl.BlockSpec((tile_r, W), in_map)]
    if has_tail:
        in_specs = [pl.BlockSpec(memory_space=pltpu.SMEM)] + data_specs
        operands = (tails, yp2, yt2)
    else:
        in_specs = data_specs
        operands = (yp2, yt2)

    out = pl.pallas_call(
        kern,
        out_shape=jax.ShapeDtypeStruct((1, 1), jnp.float32),
        grid=(n_blocks,),
        in_specs=in_specs,
        out_specs=pl.BlockSpec(memory_space=pltpu.SMEM),
        scratch_shapes=[pltpu.VMEM((8, W), jnp.float32)] * 3,
        compiler_params=pltpu.CompilerParams(
            dimension_semantics=("arbitrary",),
            vmem_limit_bytes=VMEM_LIMIT_BYTES),
        cost_estimate=pl.CostEstimate(
            flops=4 * yp2.size, transcendentals=0,
            bytes_accessed=in_bytes + 4),
    )(*operands)

    return out[0, 0]


def kernel(y_pred: jax.Array, y_true: jax.Array) -> jax.Array:
    beta2 = 1.0
    eps = 1e-6

    n = y_pred.size
    yp_flat = y_pred.reshape(-1)
    yt_flat = y_true.reshape(-1)

    n_main = (n // LANES) * LANES

    if n_main < n:
        ypt = yp_flat[n_main:].astype(jnp.float32)
        ytt = yt_flat[n_main:].astype(jnp.float32)
        tails = jnp.stack(
            [jnp.sum(ytt * ypt), jnp.sum(ypt), jnp.sum(ytt)])
    else:
        tails = None

    if n_main == 0:
        tp, sp, st = tails[0], tails[1], tails[2]
        epsf = jnp.float32(eps)
        fn = sp - tp
        fp = st - tp
        p = tp / (tp + fp + epsf)
        r = tp / (tp + fn + epsf)
        f1 = (1.0 + beta2) * (p * r) / (beta2 * p + r + epsf)
        f1 = jnp.where(jnp.isnan(f1), jnp.zeros_like(f1), f1)
        return (1.0 - f1).astype(jnp.float32)

    yp_main = yp_flat if n_main == n else yp_flat[:n_main]
    yt_main = yt_flat if n_main == n else yt_flat[:n_main]
    W = LANES
    for w in (1024, 512, 256):
        if n_main % w == 0:
            W = w
            break
    R = n_main // W
    yp2 = yp_main.reshape(R, W).astype(jnp.float32)
    yt2 = yt_main.reshape(R, W).astype(jnp.float32)
    if R % MANUAL_TILE_R == 0 and R // MANUAL_TILE_R >= MANUAL_DEPTH:
        return _soft_f1_manual(yp2, yt2, tails, beta2, eps)
    return _soft_f1_pallas(yp2, yt2, tails, beta2, eps)


# speedup vs baseline: 1.0585x; 1.0511x over previous
import functools

import jax
import jax.numpy as jnp
from jax.experimental import pallas as pl
from jax.experimental.pallas import tpu as pltpu

LANES = 128
TARGET_BLOCK_BYTES = 4 * 1024 * 1024
VMEM_LIMIT_BYTES = 48 * 1024 * 1024


def _round_up(x: int, m: int) -> int:
    return (x + m - 1) // m * m


def _fold_rows(x):
    r, w = x.shape
    return jnp.sum(x.reshape(r // 8, 8, w), axis=0)


def _soft_f1_kernel(*refs,
                    tile_r: int, n_steps: int, rows_total: int,
                    full_blocks: int, any_masked: bool, has_tail: bool,
                    beta2: float, eps: float):
    if has_tail:
        tail_ref, yp_ref, yt_ref, out_ref, tp_ref, sp_ref, st_ref = refs
    else:
        yp_ref, yt_ref, out_ref, tp_ref, sp_ref, st_ref = refs
        tail_ref = None
    s = pl.program_id(0)

    @pl.when(s == 0)
    def _init():
        tp_ref[...] = jnp.zeros_like(tp_ref)
        sp_ref[...] = jnp.zeros_like(sp_ref)
        st_ref[...] = jnp.zeros_like(st_ref)

    yp = yp_ref[...].astype(jnp.float32)
    yt = yt_ref[...].astype(jnp.float32)

    def accumulate(ypv, ytv):
        tp_ref[...] += _fold_rows(ytv * ypv)
        sp_ref[...] += _fold_rows(ypv)
        st_ref[...] += _fold_rows(ytv)

    if any_masked:
        @pl.when(s >= full_blocks)
        def _edge():
            row = s * tile_r + jax.lax.broadcasted_iota(
                jnp.int32, yp.shape, 0)
            valid = row < rows_total
            accumulate(jnp.where(valid, yp, 0.0), jnp.where(valid, yt, 0.0))

        @pl.when(s < full_blocks)
        def _full():
            accumulate(yp, yt)
    else:
        accumulate(yp, yt)

    @pl.when(s == n_steps - 1)
    def _finish():
        tp = jnp.sum(tp_ref[...])
        sp = jnp.sum(sp_ref[...])
        st = jnp.sum(st_ref[...])
        if has_tail:
            tp = tp + tail_ref[0]
            sp = sp + tail_ref[1]
            st = st + tail_ref[2]
        epsf = jnp.float32(eps)
        fn = sp - tp
        fp = st - tp
        p = tp / (tp + fp + epsf)
        r = tp / (tp + fn + epsf)
        f1 = (1.0 + beta2) * (p * r) / (beta2 * p + r + epsf)
        f1 = jnp.where(jnp.isnan(f1), jnp.zeros_like(f1), f1)
        out_ref[0, 0] = (1.0 - f1).astype(jnp.float32)


def _soft_f1_pallas(yp2: jax.Array, yt2: jax.Array, tails, beta2, eps):
    R, W = yp2.shape
    tile_r = max(8, min(_round_up(R, 8), TARGET_BLOCK_BYTES // (4 * W)))
    tile_r = _round_up(tile_r, 8)

    n_blocks = pl.cdiv(R, tile_r)
    full_blocks = R // tile_r
    any_masked = n_blocks > full_blocks

    has_tail = tails is not None
    kern = functools.partial(
        _soft_f1_kernel, tile_r=tile_r, n_steps=n_blocks,
        rows_total=R, full_blocks=full_blocks, any_masked=any_masked,
        has_tail=has_tail, beta2=beta2, eps=eps)

    in_map = lambda s: (s, 0)
    in_bytes = yp2.size * yp2.dtype.itemsize + yt2.size * yt2.dtype.itemsize

    data_specs = [pl.BlockSpec((tile_r, W), in_map),
                  pl.BlockSpec((tile_r, W), in_map)]
    if has_tail:
        in_specs = [pl.BlockSpec(memory_space=pltpu.SMEM)] + data_specs
        operands = (tails, yp2, yt2)
    else:
        in_specs = data_specs
        operands = (yp2, yt2)

    out = pl.pallas_call(
        kern,
        out_shape=jax.ShapeDtypeStruct((1, 1), jnp.float32),
        grid=(n_blocks,),
        in_specs=in_specs,
        out_specs=pl.BlockSpec(memory_space=pltpu.SMEM),
        scratch_shapes=[pltpu.VMEM((8, W), jnp.float32)] * 3,
        compiler_params=pltpu.CompilerParams(
            dimension_semantics=("arbitrary",),
            vmem_limit_bytes=VMEM_LIMIT_BYTES),
        cost_estimate=pl.CostEstimate(
            flops=4 * yp2.size, transcendentals=0,
            bytes_accessed=in_bytes + 4),
    )(*operands)

    return out[0, 0]


def kernel(y_pred: jax.Array, y_true: jax.Array) -> jax.Array:
    beta2 = 1.0
    eps = 1e-6

    n = y_pred.size
    yp_flat = y_pred.reshape(-1)
    yt_flat = y_true.reshape(-1)

    n_main = (n // LANES) * LANES

    if n_main < n:
        ypt = yp_flat[n_main:].astype(jnp.float32)
        ytt = yt_flat[n_main:].astype(jnp.float32)
        tails = jnp.stack(
            [jnp.sum(ytt * ypt), jnp.sum(ypt), jnp.sum(ytt)])
    else:
        tails = None

    if n_main == 0:
        tp, sp, st = tails[0], tails[1], tails[2]
        epsf = jnp.float32(eps)
        fn = sp - tp
        fp = st - tp
        p = tp / (tp + fp + epsf)
        r = tp / (tp + fn + epsf)
        f1 = (1.0 + beta2) * (p * r) / (beta2 * p + r + epsf)
        f1 = jnp.where(jnp.isnan(f1), jnp.zeros_like(f1), f1)
        return (1.0 - f1).astype(jnp.float32)

    yp_main = yp_flat if n_main == n else yp_flat[:n_main]
    yt_main = yt_flat if n_main == n else yt_flat[:n_main]
    W = LANES
    for w in (1024, 512, 256):
        if n_main % w == 0:
            W = w
            break
    R = n_main // W
    return _soft_f1_pallas(
        yp_main.reshape(R, W), yt_main.reshape(R, W), tails, beta2, eps)
